# baseline (speedup 1.0000x reference)
import functools

import jax
import jax.numpy as jnp
from jax import lax
from jax.experimental import pallas as pl
from jax.experimental.pallas import tpu as pltpu
from jax.experimental.pallas import tpu_sc as plsc

B = 16384
D = 128
NC = 2
NS = 16
L = 16
NW = NC * NS
BPW = B // NW
CH = 128
NCH = BPW // CH


def _sc_body(idx_hbm, ent_hbm, path_hbm, out_hbm,
             idx_path_v, idx_pos_v, idx_neg_v,
             pos_v, neg_v, path_v, out_v,
             sem0, sem1, sem2):
    w = lax.axis_index("s") * NC + lax.axis_index("c")
    base = w * BPW
    pltpu.sync_copy(idx_hbm.at[0, w], idx_path_v)
    pltpu.sync_copy(idx_hbm.at[1, w], idx_pos_v)
    pltpu.sync_copy(idx_hbm.at[2, w], idx_neg_v)

    for c in range(NCH):
        cp = pltpu.async_copy(ent_hbm.at[idx_pos_v.at[c]], pos_v, sem0)
        cn = pltpu.async_copy(ent_hbm.at[idx_neg_v.at[c]], neg_v, sem1)
        ct = pltpu.async_copy(path_hbm.at[idx_path_v.at[c]], path_v, sem2)
        cp.wait()
        cn.wait()
        ct.wait()

        def group_body(g, _, c=c):
            rows = lax.iota(jnp.int32, L) + g * L

            def dim_body(dd, acc):
                dsp = jnp.full((L,), 0, jnp.int32) + dd
                p = plsc.load_gather(pos_v, [rows, dsp])
                n = plsc.load_gather(neg_v, [rows, dsp])
                t = plsc.load_gather(path_v, [rows, dsp])
                return acc + (p - n) * t

            acc = lax.fori_loop(0, D, dim_body, jnp.zeros((L,), jnp.float32))
            out_v[pl.ds(c * CH + g * L, L)] = acc
            return 0

        lax.fori_loop(0, CH // L, group_body, 0)

    pltpu.sync_copy(out_v, out_hbm.at[pl.ds(base, BPW)])


@functools.partial(
    pl.kernel,
    out_type=jax.ShapeDtypeStruct((B,), jnp.float32),
    mesh=plsc.VectorSubcoreMesh(core_axis_name="c", subcore_axis_name="s"),
    compiler_params=pltpu.CompilerParams(needs_layout_passes=False),
    scratch_types=[
        pltpu.VMEM((NCH, CH), jnp.int32),
        pltpu.VMEM((NCH, CH), jnp.int32),
        pltpu.VMEM((NCH, CH), jnp.int32),
        pltpu.VMEM((CH, D), jnp.float32),
        pltpu.VMEM((CH, D), jnp.float32),
        pltpu.VMEM((CH, D), jnp.float32),
        pltpu.VMEM((BPW,), jnp.float32),
        pltpu.SemaphoreType.DMA,
        pltpu.SemaphoreType.DMA,
        pltpu.SemaphoreType.DMA,
    ],
)
def _uni_model_sc(idx_hbm, ent_hbm, path_hbm, out_hbm, *rest):
    _sc_body(idx_hbm, ent_hbm, path_hbm, out_hbm, *rest)


@jax.jit
def kernel(ents_path_idxs, ent_table, path_table):
    idxs = ents_path_idxs.astype(jnp.int32).T.reshape(3, NW, NCH, CH)
    out = _uni_model_sc(idxs, ent_table, path_table)
    return out.reshape(B, 1, 1)


# speedup vs baseline: 1.1829x; 1.1829x over previous
import functools

import jax
import jax.numpy as jnp
from jax import lax
from jax.experimental import pallas as pl
from jax.experimental.pallas import tpu as pltpu
from jax.experimental.pallas import tpu_sc as plsc

B = 16384
D = 128
NC = 2
NS = 16
L = 16
NW = NC * NS
BPW = B // NW
CH = 128
NCH = BPW // CH
UD = 16
NACC = 4


def _sc_body(idx_hbm, ent_hbm, path_hbm, out_hbm,
             idx_path_v, idx_pos_v, idx_neg_v,
             pos0, neg0, path0, pos1, neg1, path1, out_v,
             sa0, sa1, sa2, sb0, sb1, sb2):
    w = lax.axis_index("s") * NC + lax.axis_index("c")
    base = w * BPW
    pltpu.sync_copy(idx_hbm.at[0, w], idx_path_v)
    pltpu.sync_copy(idx_hbm.at[1, w], idx_pos_v)
    pltpu.sync_copy(idx_hbm.at[2, w], idx_neg_v)

    bufs = ((pos0, neg0, path0, sa0, sa1, sa2),
            (pos1, neg1, path1, sb0, sb1, sb2))

    def issue(c, pos_b, neg_b, path_b, s0, s1, s2):
        return (pltpu.async_copy(ent_hbm.at[idx_pos_v.at[c]], pos_b, s0),
                pltpu.async_copy(ent_hbm.at[idx_neg_v.at[c]], neg_b, s1),
                pltpu.async_copy(path_hbm.at[idx_path_v.at[c]], path_b, s2))

    def compute(c, pos_b, neg_b, path_b, *_):
        def group_body(g, _):
            rows = lax.iota(jnp.int32, L) + g * L

            def block_body(bb, accs):
                d0 = bb * UD
                accs = list(accs)
                for k in range(UD):
                    dsp = jnp.full((L,), 0, jnp.int32) + (d0 + k)
                    p = plsc.load_gather(pos_b, [rows, dsp])
                    n = plsc.load_gather(neg_b, [rows, dsp])
                    t = plsc.load_gather(path_b, [rows, dsp])
                    accs[k % NACC] = accs[k % NACC] + (p - n) * t
                return tuple(accs)

            zero = jnp.zeros((L,), jnp.float32)
            accs = lax.fori_loop(0, D // UD, block_body, (zero,) * NACC)
            acc = (accs[0] + accs[1]) + (accs[2] + accs[3])
            out_v[pl.ds(c * CH + g * L, L)] = acc
            return 0

        lax.fori_loop(0, CH // L, group_body, 0)

    pending = {0: issue(0, *bufs[0])}
    for c in range(NCH):
        if c + 1 < NCH:
            pending[c + 1] = issue(c + 1, *bufs[(c + 1) % 2])
        for desc in pending.pop(c):
            desc.wait()
        compute(c, *bufs[c % 2])

    pltpu.sync_copy(out_v, out_hbm.at[pl.ds(base, BPW)])


@functools.partial(
    pl.kernel,
    out_type=jax.ShapeDtypeStruct((B,), jnp.float32),
    mesh=plsc.VectorSubcoreMesh(core_axis_name="c", subcore_axis_name="s"),
    compiler_params=pltpu.CompilerParams(needs_layout_passes=False),
    scratch_types=[
        pltpu.VMEM((NCH, CH), jnp.int32),
        pltpu.VMEM((NCH, CH), jnp.int32),
        pltpu.VMEM((NCH, CH), jnp.int32),
        pltpu.VMEM((CH, D), jnp.float32),
        pltpu.VMEM((CH, D), jnp.float32),
        pltpu.VMEM((CH, D), jnp.float32),
        pltpu.VMEM((CH, D), jnp.float32),
        pltpu.VMEM((CH, D), jnp.float32),
        pltpu.VMEM((CH, D), jnp.float32),
        pltpu.VMEM((BPW,), jnp.float32),
        pltpu.SemaphoreType.DMA,
        pltpu.SemaphoreType.DMA,
        pltpu.SemaphoreType.DMA,
        pltpu.SemaphoreType.DMA,
        pltpu.SemaphoreType.DMA,
        pltpu.SemaphoreType.DMA,
    ],
)
def _uni_model_sc(idx_hbm, ent_hbm, path_hbm, out_hbm, *rest):
    _sc_body(idx_hbm, ent_hbm, path_hbm, out_hbm, *rest)


@jax.jit
def kernel(ents_path_idxs, ent_table, path_table):
    idxs = ents_path_idxs.astype(jnp.int32).T.reshape(3, NW, NCH, CH)
    out = _uni_model_sc(idxs, ent_table, path_table)
    return out.reshape(B, 1, 1)


# speedup vs baseline: 3.7479x; 3.1684x over previous
import functools

import jax
import jax.numpy as jnp
from jax import lax
from jax.experimental import pallas as pl
from jax.experimental.pallas import tpu as pltpu
from jax.experimental.pallas import tpu_sc as plsc

B = 16384
D = 128
NC = 2
NS = 16
L = 16
NW = NC * NS
BPW = B // NW
CH = 128
NCH = BPW // CH
UD = 16
NACC = 4


def _sc_body(idx_hbm, ent_hbm, path_hbm, out_hbm,
             idx_path_v, idx_pos_v, idx_neg_v,
             pos0, neg0, path0, pos1, neg1, path1, out_v,
             sa0, sa1, sa2, sb0, sb1, sb2):
    w = lax.axis_index("s") * NC + lax.axis_index("c")
    base = w * BPW
    pltpu.sync_copy(idx_hbm.at[0, w], idx_path_v)
    pltpu.sync_copy(idx_hbm.at[1, w], idx_pos_v)
    pltpu.sync_copy(idx_hbm.at[2, w], idx_neg_v)

    bufs = ((pos0, neg0, path0, sa0, sa1, sa2),
            (pos1, neg1, path1, sb0, sb1, sb2))

    def issue(c, pos_b, neg_b, path_b, s0, s1, s2):
        return (pltpu.async_copy(ent_hbm.at[idx_pos_v.at[c]], pos_b, s0),
                pltpu.async_copy(ent_hbm.at[idx_neg_v.at[c]], neg_b, s1),
                pltpu.async_copy(path_hbm.at[idx_path_v.at[c]], path_b, s2))

    def compute(c, pos_b, neg_b, path_b, *_):
        lane = lax.iota(jnp.int32, L)

        def group_body(g, _):
            rows = lane + g * L

            def block_body(bb, accs):
                d0 = bb * UD
                accs = list(accs)
                for k in range(UD):
                    dsp = (lane + (d0 + k)) & (D - 1)
                    p = plsc.load_gather(pos_b, [rows, dsp])
                    n = plsc.load_gather(neg_b, [rows, dsp])
                    t = plsc.load_gather(path_b, [rows, dsp])
                    accs[k % NACC] = accs[k % NACC] + (p - n) * t
                return tuple(accs)

            zero = jnp.zeros((L,), jnp.float32)
            accs = lax.fori_loop(0, D // UD, block_body, (zero,) * NACC)
            acc = (accs[0] + accs[1]) + (accs[2] + accs[3])
            out_v[pl.ds(c * CH + g * L, L)] = acc
            return 0

        lax.fori_loop(0, CH // L, group_body, 0)

    pending = {0: issue(0, *bufs[0])}
    for c in range(NCH):
        if c + 1 < NCH:
            pending[c + 1] = issue(c + 1, *bufs[(c + 1) % 2])
        for desc in pending.pop(c):
            desc.wait()
        compute(c, *bufs[c % 2])

    pltpu.sync_copy(out_v, out_hbm.at[pl.ds(base, BPW)])


@functools.partial(
    pl.kernel,
    out_type=jax.ShapeDtypeStruct((B,), jnp.float32),
    mesh=plsc.VectorSubcoreMesh(core_axis_name="c", subcore_axis_name="s"),
    compiler_params=pltpu.CompilerParams(needs_layout_passes=False),
    scratch_types=[
        pltpu.VMEM((NCH, CH), jnp.int32),
        pltpu.VMEM((NCH, CH), jnp.int32),
        pltpu.VMEM((NCH, CH), jnp.int32),
        pltpu.VMEM((CH, D), jnp.float32),
        pltpu.VMEM((CH, D), jnp.float32),
        pltpu.VMEM((CH, D), jnp.float32),
        pltpu.VMEM((CH, D), jnp.float32),
        pltpu.VMEM((CH, D), jnp.float32),
        pltpu.VMEM((CH, D), jnp.float32),
        pltpu.VMEM((BPW,), jnp.float32),
        pltpu.SemaphoreType.DMA,
        pltpu.SemaphoreType.DMA,
        pltpu.SemaphoreType.DMA,
        pltpu.SemaphoreType.DMA,
        pltpu.SemaphoreType.DMA,
        pltpu.SemaphoreType.DMA,
    ],
)
def _uni_model_sc(idx_hbm, ent_hbm, path_hbm, out_hbm, *rest):
    _sc_body(idx_hbm, ent_hbm, path_hbm, out_hbm, *rest)


@jax.jit
def kernel(ents_path_idxs, ent_table, path_table):
    idxs = ents_path_idxs.astype(jnp.int32).T.reshape(3, NW, NCH, CH)
    out = _uni_model_sc(idxs, ent_table, path_table)
    return out.reshape(B, 1, 1)
